# baseline (speedup 1.0000x reference)
import functools

import jax
import jax.numpy as jnp
from jax import lax
from jax.experimental import pallas as pl
from jax.experimental.pallas import tpu as pltpu
from jax.experimental.pallas import tpu_sc as plsc

_NUM_EMBED = 1024
_DIM = 64
_N = 16 * 1024
_NC = 2
_NS = 16
_NW = _NC * _NS
_ROWS_PER_W = _N // _NW
_GCHUNK = 128


def _dist_body(x_ref, xp_ref, cb_ref, idx_ref, loss_ref, s_buf, *,
               block_rows, num_blocks, total_elems):
    i = pl.program_id(0)
    cb = cb_ref[...]

    s_buf[i % 2] = jax.lax.dot_general(
        x_ref[...], cb, (((1,), (1,)), ((), ())),
        preferred_element_type=jnp.float32)

    x = xp_ref[...]
    s = s_buf[(i - 1) % 2]
    rs = jnp.sum(x * x, axis=1, keepdims=True)
    c = jnp.sum(cb * cb, axis=1)
    d = (rs + c[None, :]) - 2.0 * s
    m = jnp.min(d, axis=1, keepdims=True)
    iota = jax.lax.broadcasted_iota(
        jnp.int32, (block_rows, _NUM_EMBED), 1)
    idx = jnp.min(jnp.where(d == m, iota, _NUM_EMBED), axis=1,
                  keepdims=True)
    idx_ref[...] = idx
    part = jnp.reshape(jnp.sum(m), (1, 1))

    @pl.when(i == 1)
    def _():
        loss_ref[...] = part

    @pl.when(i > 1)
    def _():
        loss_ref[...] += part

    @pl.when(i == num_blocks)
    def _():
        loss_ref[...] = loss_ref[...] * (1.25 / total_elems)


_sc_mesh = plsc.VectorSubcoreMesh(core_axis_name="c", subcore_axis_name="s")


@functools.partial(
    pl.kernel,
    mesh=_sc_mesh,
    compiler_params=pltpu.CompilerParams(use_tc_tiling_on_sc=False),
    out_type=jax.ShapeDtypeStruct((_N, _DIM), jnp.float32),
    scratch_types=[
        pltpu.VMEM((_ROWS_PER_W,), jnp.int32),
        pltpu.VMEM((_ROWS_PER_W, _DIM), jnp.float32),
        pltpu.SemaphoreType.DMA,
    ],
)
def _sc_gather(cb_hbm, idx_hbm, out_hbm, idx_v, rows_v, sem):
    wid = lax.axis_index("s") * _NC + lax.axis_index("c")
    base = wid * _ROWS_PER_W
    pltpu.sync_copy(idx_hbm.at[pl.ds(base, _ROWS_PER_W)], idx_v)
    copies = []
    for k in range(_ROWS_PER_W // _GCHUNK):
        copies.append(pltpu.async_copy(
            cb_hbm.at[idx_v.at[pl.ds(k * _GCHUNK, _GCHUNK)]],
            rows_v.at[pl.ds(k * _GCHUNK, _GCHUNK)],
            sem))
    for c in copies:
        c.wait()
    pltpu.sync_copy(rows_v, out_hbm.at[pl.ds(base, _ROWS_PER_W)])


def kernel(inputs, codebook):
    input_shape = inputs.shape
    x = inputs.reshape(-1, _DIM)
    n = x.shape[0]
    block_rows = 2048
    num_blocks = n // block_rows
    nb = num_blocks
    body = functools.partial(
        _dist_body, block_rows=block_rows, num_blocks=num_blocks,
        total_elems=float(inputs.size))
    idx, loss = pl.pallas_call(
        body,
        grid=(num_blocks + 1,),
        in_specs=[
            pl.BlockSpec((block_rows, _DIM),
                         lambda i: (jnp.minimum(i, nb - 1), 0)),
            pl.BlockSpec((block_rows, _DIM),
                         lambda i: (jnp.maximum(i - 1, 0), 0)),
            pl.BlockSpec((_NUM_EMBED, _DIM), lambda i: (0, 0)),
        ],
        out_specs=[
            pl.BlockSpec((block_rows, 1),
                         lambda i: (jnp.maximum(i - 1, 0), 0)),
            pl.BlockSpec((1, 1), lambda i: (0, 0)),
        ],
        out_shape=[
            jax.ShapeDtypeStruct((n, 1), jnp.int32),
            jax.ShapeDtypeStruct((1, 1), jnp.float32),
        ],
        scratch_shapes=[
            pltpu.VMEM((2, block_rows, _NUM_EMBED), jnp.float32),
        ],
    )(x, x, codebook)
    q = _sc_gather(codebook, idx.reshape(n))
    return q.reshape(input_shape), loss[0, 0]


# speedup vs baseline: 1.1652x; 1.1652x over previous
import functools

import jax
import jax.numpy as jnp
from jax import lax
from jax.experimental import pallas as pl
from jax.experimental.pallas import tpu as pltpu
from jax.experimental.pallas import tpu_sc as plsc

_NUM_EMBED = 1024
_DIM = 64
_N = 16 * 1024
_NC = 2
_NS = 16
_NW = _NC * _NS
_ROWS_PER_W = _N // _NW
_GCHUNK = 128


def _dist_body(x_ref, cb_ref, idx_ref, loss_ref, *,
               block_rows, num_blocks, total_elems):
    i = pl.program_id(0)
    x = x_ref[...]
    cb = cb_ref[...]
    s = jax.lax.dot_general(
        x, cb, (((1,), (1,)), ((), ())),
        preferred_element_type=jnp.float32)
    rs = jnp.sum(x * x, axis=1, keepdims=True)
    c = jnp.sum(cb * cb, axis=1)
    d = (rs + c[None, :]) - 2.0 * s
    m = jnp.min(d, axis=1, keepdims=True)
    iota = jax.lax.broadcasted_iota(
        jnp.int32, (block_rows, _NUM_EMBED), 1)
    idx = jnp.min(jnp.where(d == m, iota, _NUM_EMBED), axis=1,
                  keepdims=True)
    idx_ref[...] = idx
    part = jnp.reshape(jnp.sum(m), (1, 1))

    @pl.when(i == 0)
    def _():
        loss_ref[...] = part

    @pl.when(i > 0)
    def _():
        loss_ref[...] += part

    @pl.when(i == num_blocks - 1)
    def _():
        loss_ref[...] = loss_ref[...] * (1.25 / total_elems)


_sc_mesh = plsc.VectorSubcoreMesh(core_axis_name="c", subcore_axis_name="s")


@functools.partial(
    pl.kernel,
    mesh=_sc_mesh,
    compiler_params=pltpu.CompilerParams(use_tc_tiling_on_sc=False),
    out_type=jax.ShapeDtypeStruct((_N, _DIM), jnp.float32),
    scratch_types=[
        pltpu.VMEM((_ROWS_PER_W,), jnp.int32),
        pltpu.VMEM((_ROWS_PER_W, _DIM), jnp.float32),
        pltpu.SemaphoreType.DMA,
    ],
)
def _sc_gather(cb_hbm, idx_hbm, out_hbm, idx_v, rows_v, sem):
    wid = lax.axis_index("s") * _NC + lax.axis_index("c")
    base = wid * _ROWS_PER_W
    pltpu.sync_copy(idx_hbm.at[pl.ds(base, _ROWS_PER_W)], idx_v)
    copies = []
    for k in range(_ROWS_PER_W // _GCHUNK):
        copies.append(pltpu.async_copy(
            cb_hbm.at[idx_v.at[pl.ds(k * _GCHUNK, _GCHUNK)]],
            rows_v.at[pl.ds(k * _GCHUNK, _GCHUNK)],
            sem))
    for c in copies:
        c.wait()
    pltpu.sync_copy(rows_v, out_hbm.at[pl.ds(base, _ROWS_PER_W)])


def kernel(inputs, codebook):
    input_shape = inputs.shape
    x = inputs.reshape(-1, _DIM)
    n = x.shape[0]
    block_rows = 2048
    num_blocks = n // block_rows
    nb = num_blocks
    body = functools.partial(
        _dist_body, block_rows=block_rows, num_blocks=num_blocks,
        total_elems=float(inputs.size))
    idx, loss = pl.pallas_call(
        body,
        grid=(num_blocks,),
        in_specs=[
            pl.BlockSpec((block_rows, _DIM), lambda i: (i, 0)),
            pl.BlockSpec((_NUM_EMBED, _DIM), lambda i: (0, 0)),
        ],
        out_specs=[
            pl.BlockSpec((block_rows, 1), lambda i: (i, 0)),
            pl.BlockSpec((1, 1), lambda i: (0, 0)),
        ],
        out_shape=[
            jax.ShapeDtypeStruct((n, 1), jnp.int32),
            jax.ShapeDtypeStruct((1, 1), jnp.float32),
        ],
    )(x, codebook)
    q = _sc_gather(codebook, idx.reshape(n))
    return q.reshape(input_shape), loss[0, 0]


# speedup vs baseline: 1.4506x; 1.2449x over previous
import functools

import jax
import jax.numpy as jnp
from jax.experimental import pallas as pl

_NUM_EMBED = 1024
_DIM = 64


def _vq_body(x_ref, cb_ref, q_ref, loss_ref, *, block_rows, num_blocks,
             total_elems):
    i = pl.program_id(0)
    x = x_ref[...]
    cb = cb_ref[...]
    s = jax.lax.dot_general(
        x, cb, (((1,), (1,)), ((), ())),
        preferred_element_type=jnp.float32)
    rs = jnp.sum(x * x, axis=1, keepdims=True)
    c = jnp.sum(cb * cb, axis=1)
    d = (rs + c[None, :]) - 2.0 * s
    m = jnp.min(d, axis=1, keepdims=True)
    iota = jax.lax.broadcasted_iota(
        jnp.int32, (block_rows, _NUM_EMBED), 1)
    idx = jnp.min(jnp.where(d == m, iota, _NUM_EMBED), axis=1,
                  keepdims=True)
    onehot = (iota == idx).astype(jnp.float32)
    q_ref[...] = jax.lax.dot_general(
        onehot, cb, (((1,), (0,)), ((), ())),
        preferred_element_type=jnp.float32)
    part = jnp.reshape(jnp.sum(m), (1, 1))

    @pl.when(i == 0)
    def _():
        loss_ref[...] = part

    @pl.when(i > 0)
    def _():
        loss_ref[...] += part

    @pl.when(i == num_blocks - 1)
    def _():
        loss_ref[...] = loss_ref[...] * (1.25 / total_elems)


def kernel(inputs, codebook):
    input_shape = inputs.shape
    x = inputs.reshape(-1, _DIM)
    n = x.shape[0]
    block_rows = 2048
    num_blocks = n // block_rows
    body = functools.partial(
        _vq_body, block_rows=block_rows, num_blocks=num_blocks,
        total_elems=float(inputs.size))
    q, loss = pl.pallas_call(
        body,
        grid=(num_blocks,),
        in_specs=[
            pl.BlockSpec((block_rows, _DIM), lambda i: (i, 0)),
            pl.BlockSpec((_NUM_EMBED, _DIM), lambda i: (0, 0)),
        ],
        out_specs=[
            pl.BlockSpec((block_rows, _DIM), lambda i: (i, 0)),
            pl.BlockSpec((1, 1), lambda i: (0, 0)),
        ],
        out_shape=[
            jax.ShapeDtypeStruct((n, _DIM), jnp.float32),
            jax.ShapeDtypeStruct((1, 1), jnp.float32),
        ],
    )(x, codebook)
    return q.reshape(input_shape), loss[0, 0]


# speedup vs baseline: 1.4759x; 1.0175x over previous
import functools

import jax
import jax.numpy as jnp
from jax.experimental import pallas as pl

_NUM_EMBED = 1024
_DIM = 64


def _vq_body(x_ref, cb_ref, q_ref, loss_ref, *, block_rows, num_blocks,
             total_elems):
    i = pl.program_id(0)
    x = x_ref[...]
    cb = cb_ref[...]
    s = jax.lax.dot_general(
        x, cb, (((1,), (1,)), ((), ())),
        preferred_element_type=jnp.float32)
    rs = jnp.sum(x * x, axis=1, keepdims=True)
    c = jnp.sum(cb * cb, axis=1)
    d = (rs + c[None, :]) - 2.0 * s
    m = jnp.min(d, axis=1, keepdims=True)
    iota = jax.lax.broadcasted_iota(
        jnp.int32, (block_rows, _NUM_EMBED), 1)
    idx = jnp.min(jnp.where(d == m, iota, _NUM_EMBED), axis=1,
                  keepdims=True)
    onehot = (iota == idx).astype(jnp.bfloat16)
    q_ref[...] = jax.lax.dot_general(
        onehot, cb.astype(jnp.bfloat16), (((1,), (0,)), ((), ())),
        preferred_element_type=jnp.float32)
    part = jnp.reshape(jnp.sum(m), (1, 1))

    @pl.when(i == 0)
    def _():
        loss_ref[...] = part

    @pl.when(i > 0)
    def _():
        loss_ref[...] += part

    @pl.when(i == num_blocks - 1)
    def _():
        loss_ref[...] = loss_ref[...] * (1.25 / total_elems)


def kernel(inputs, codebook):
    input_shape = inputs.shape
    x = inputs.reshape(-1, _DIM)
    n = x.shape[0]
    block_rows = 4096
    num_blocks = n // block_rows
    body = functools.partial(
        _vq_body, block_rows=block_rows, num_blocks=num_blocks,
        total_elems=float(inputs.size))
    q, loss = pl.pallas_call(
        body,
        grid=(num_blocks,),
        in_specs=[
            pl.BlockSpec((block_rows, _DIM), lambda i: (i, 0)),
            pl.BlockSpec((_NUM_EMBED, _DIM), lambda i: (0, 0)),
        ],
        out_specs=[
            pl.BlockSpec((block_rows, _DIM), lambda i: (i, 0)),
            pl.BlockSpec((1, 1), lambda i: (0, 0)),
        ],
        out_shape=[
            jax.ShapeDtypeStruct((n, _DIM), jnp.float32),
            jax.ShapeDtypeStruct((1, 1), jnp.float32),
        ],
    )(x, codebook)
    return q.reshape(input_shape), loss[0, 0]
